# Initial kernel scaffold; baseline (speedup 1.0000x reference)
import jax
import jax.numpy as jnp
from jax.experimental import pallas as pl


def kernel(token, h_pos, t_pos, word_emb, h_pos_emb, t_pos_emb):
    raise NotImplementedError("write your pallas kernel here")


# speedup vs baseline: 7.7592x; 7.7592x over previous
import functools

import jax
import jax.numpy as jnp
from jax import lax
from jax.experimental import pallas as pl
from jax.experimental.pallas import tpu as pltpu
from jax.experimental.pallas import tpu_sc as plsc

_WORD_DIM = 50
_POS_DIM = 5
_OUT_DIM = 60
_PAD_DIM = 64
_CPY_DIM = 56
_NC, _NSUB = 2, 16
_NW = _NC * _NSUB
_TB = 512
_IB = 128


@functools.lru_cache(maxsize=None)
def _build(n_rows: int, pos_vocab: int):
    n_blk = _TB // _IB
    rw = n_rows // _NW
    n_chunk = rw // _TB
    assert rw * _NW == n_rows and n_chunk * _TB == rw

    mesh = plsc.VectorSubcoreMesh(
        core_axis_name="c", subcore_axis_name="s",
        num_cores=_NC, num_subcores=_NSUB)

    @functools.partial(
        pl.kernel,
        out_type=jax.ShapeDtypeStruct((n_rows, _OUT_DIM), jnp.float32),
        mesh=mesh,
        compiler_params=pltpu.CompilerParams(
            needs_layout_passes=False, use_tc_tiling_on_sc=False),
        scratch_types=[
            pltpu.VMEM((n_blk, _IB), jnp.int32),
            pltpu.VMEM((_TB,), jnp.int32),
            pltpu.VMEM((_TB,), jnp.int32),
            pltpu.VMEM((_TB, _PAD_DIM), jnp.float32),
            pltpu.VMEM((_TB, _OUT_DIM - _CPY_DIM), jnp.float32),
            pltpu.VMEM((pos_vocab * _POS_DIM,), jnp.float32),
            pltpu.VMEM((pos_vocab * _POS_DIM,), jnp.float32),
            pltpu.SemaphoreType.DMA,
        ],
    )
    def sc_embed(wpad, tok2, hflat, tflat, htab, ttab, out,
                 tok_v, h_v, t_v, word_v, tail_v, htab_v, ttab_v, sem):
        wid = lax.axis_index("s") * _NC + lax.axis_index("c")
        pltpu.sync_copy(htab, htab_v)
        pltpu.sync_copy(ttab, ttab_v)
        blk0 = wid * (rw // _IB)

        def chunk(c, _):
            blk = blk0 + c * n_blk
            row0 = blk * _IB
            pltpu.sync_copy(tok2.at[pl.ds(blk, n_blk)], tok_v)
            pltpu.sync_copy(hflat.at[pl.ds(row0, _TB)], h_v)
            pltpu.sync_copy(tflat.at[pl.ds(row0, _TB)], t_v)
            cps = [
                pltpu.async_copy(wpad.at[tok_v.at[j]],
                                 word_v.at[pl.ds(j * _IB, _IB)], sem)
                for j in range(n_blk)
            ]
            for cp in cps:
                cp.wait()

            def fill(i, _):
                h5 = h_v[pl.ds(i * 16, 16)] * _POS_DIM
                t5 = t_v[pl.ds(i * 16, 16)] * _POS_DIM
                rows = i * 16 + lax.iota(jnp.int32, 16)
                for d in range(_POS_DIM):
                    hv = plsc.load_gather(htab_v, [h5 + d])
                    plsc.store_scatter(
                        word_v,
                        [rows, jnp.full((16,), _WORD_DIM + d, jnp.int32)], hv)
                for d in range(_POS_DIM):
                    tv = plsc.load_gather(ttab_v, [t5 + d])
                    if d == 0:
                        plsc.store_scatter(
                            word_v,
                            [rows,
                             jnp.full((16,), _WORD_DIM + _POS_DIM, jnp.int32)],
                            tv)
                    else:
                        plsc.store_scatter(
                            tail_v,
                            [rows, jnp.full((16,), d - 1, jnp.int32)], tv)
                return 0

            lax.fori_loop(0, _TB // 16, fill, 0)
            pltpu.sync_copy(word_v.at[:, pl.ds(0, _CPY_DIM)],
                            out.at[pl.ds(row0, _TB), pl.ds(0, _CPY_DIM)])
            pltpu.sync_copy(tail_v,
                            out.at[pl.ds(row0, _TB),
                                   pl.ds(_CPY_DIM, _OUT_DIM - _CPY_DIM)])
            return 0

        lax.fori_loop(0, n_chunk, chunk, 0)

    return sc_embed


def kernel(token, h_pos, t_pos, word_emb, h_pos_emb, t_pos_emb):
    b, ns, l = token.shape
    n_rows = b * ns * l
    vocab = word_emb.shape[0]
    pos_vocab = h_pos_emb.shape[0]

    wpad = jnp.concatenate(
        [word_emb, jnp.zeros((vocab, _PAD_DIM - _WORD_DIM), jnp.float32)],
        axis=1)
    tok2 = token.reshape(n_rows // _IB, _IB)
    hflat = h_pos.reshape(n_rows)
    tflat = t_pos.reshape(n_rows)

    out = _build(n_rows, pos_vocab)(
        wpad, tok2, hflat, tflat,
        h_pos_emb.reshape(-1), t_pos_emb.reshape(-1))
    return out.reshape(b, ns, l, _OUT_DIM)


# speedup vs baseline: 8.7245x; 1.1244x over previous
import functools

import jax
import jax.numpy as jnp
from jax import lax
from jax.experimental import pallas as pl
from jax.experimental.pallas import tpu as pltpu
from jax.experimental.pallas import tpu_sc as plsc

_WORD_DIM = 50
_POS_DIM = 5
_OUT_DIM = 60
_PAD_DIM = 64
_CPY_DIM = 56
_NC, _NSUB = 2, 16
_NW = _NC * _NSUB
_TB = 512
_IB = 128


@functools.lru_cache(maxsize=None)
def _build(n_rows: int, pos_vocab: int):
    n_blk = _TB // _IB
    rw = n_rows // _NW
    n_chunk = rw // _TB
    assert rw * _NW == n_rows and n_chunk * _TB == rw
    assert n_chunk % 2 == 0

    mesh = plsc.VectorSubcoreMesh(
        core_axis_name="c", subcore_axis_name="s",
        num_cores=_NC, num_subcores=_NSUB)

    tail_w = _OUT_DIM - _CPY_DIM

    @functools.partial(
        pl.kernel,
        out_type=jax.ShapeDtypeStruct((n_rows, _OUT_DIM), jnp.float32),
        mesh=mesh,
        compiler_params=pltpu.CompilerParams(
            needs_layout_passes=False, use_tc_tiling_on_sc=False),
        scratch_types=[
            pltpu.VMEM((n_blk, _IB), jnp.int32),
            pltpu.VMEM((n_blk, _IB), jnp.int32),
            pltpu.VMEM((_TB,), jnp.int32),
            pltpu.VMEM((_TB,), jnp.int32),
            pltpu.VMEM((_TB,), jnp.int32),
            pltpu.VMEM((_TB,), jnp.int32),
            pltpu.VMEM((_TB, _PAD_DIM), jnp.float32),
            pltpu.VMEM((_TB, _PAD_DIM), jnp.float32),
            pltpu.VMEM((_TB, tail_w), jnp.float32),
            pltpu.VMEM((_TB, tail_w), jnp.float32),
            pltpu.VMEM((pos_vocab * _POS_DIM,), jnp.float32),
            pltpu.VMEM((pos_vocab * _POS_DIM,), jnp.float32),
            pltpu.SemaphoreType.DMA,
            pltpu.SemaphoreType.DMA,
            pltpu.SemaphoreType.DMA,
            pltpu.SemaphoreType.DMA,
        ],
    )
    def sc_embed(wpad, tok2, hflat, tflat, htab, ttab, out,
                 tok_a, tok_b, h_a, h_b, t_a, t_b,
                 word_a, word_b, tail_a, tail_b, htab_v, ttab_v,
                 gsem_a, gsem_b, wsem_a, wsem_b):
        wid = lax.axis_index("s") * _NC + lax.axis_index("c")
        pltpu.sync_copy(htab, htab_v)
        pltpu.sync_copy(ttab, ttab_v)
        blk0 = wid * (rw // _IB)

        def load_idx(c, tok_v, h_v, t_v):
            blk = blk0 + c * n_blk
            row0 = blk * _IB
            pltpu.sync_copy(tok2.at[pl.ds(blk, n_blk)], tok_v)
            pltpu.sync_copy(hflat.at[pl.ds(row0, _TB)], h_v)
            pltpu.sync_copy(tflat.at[pl.ds(row0, _TB)], t_v)

        def fire_gather(tok_v, word_v, gsem):
            for j in range(n_blk):
                pltpu.async_copy(wpad.at[tok_v.at[j]],
                                 word_v.at[pl.ds(j * _IB, _IB)], gsem)

        def wait_gather(tok_v, word_v, gsem):
            for j in range(n_blk):
                pltpu.make_async_copy(
                    wpad.at[tok_v.at[j]],
                    word_v.at[pl.ds(j * _IB, _IB)], gsem).wait()

        def fill(h_v, t_v, word_v, tail_v):
            def body(i, _):
                h5 = h_v[pl.ds(i * 16, 16)] * _POS_DIM
                t5 = t_v[pl.ds(i * 16, 16)] * _POS_DIM
                rows = i * 16 + lax.iota(jnp.int32, 16)
                for d in range(_POS_DIM):
                    hv = plsc.load_gather(htab_v, [h5 + d])
                    plsc.store_scatter(
                        word_v,
                        [rows, jnp.full((16,), _WORD_DIM + d, jnp.int32)], hv)
                for d in range(_POS_DIM):
                    tv = plsc.load_gather(ttab_v, [t5 + d])
                    if d == 0:
                        plsc.store_scatter(
                            word_v,
                            [rows,
                             jnp.full((16,), _WORD_DIM + _POS_DIM, jnp.int32)],
                            tv)
                    else:
                        plsc.store_scatter(
                            tail_v,
                            [rows, jnp.full((16,), d - 1, jnp.int32)], tv)
                return 0

            lax.fori_loop(0, _TB // 16, body, 0)

        def out_slices(c):
            row0 = (blk0 + c * n_blk) * _IB
            return (out.at[pl.ds(row0, _TB), pl.ds(0, _CPY_DIM)],
                    out.at[pl.ds(row0, _TB), pl.ds(_CPY_DIM, tail_w)])

        def fire_write(c, word_v, tail_v, wsem):
            main_dst, tail_dst = out_slices(c)
            pltpu.async_copy(word_v.at[:, pl.ds(0, _CPY_DIM)], main_dst, wsem)
            pltpu.async_copy(tail_v, tail_dst, wsem)

        def wait_write(c, word_v, tail_v, wsem):
            main_dst, tail_dst = out_slices(c)
            pltpu.make_async_copy(
                word_v.at[:, pl.ds(0, _CPY_DIM)], main_dst, wsem).wait()
            pltpu.make_async_copy(tail_v, tail_dst, wsem).wait()

        bufs = (
            (tok_a, h_a, t_a, word_a, tail_a, gsem_a, wsem_a),
            (tok_b, h_b, t_b, word_b, tail_b, gsem_b, wsem_b),
        )

        load_idx(0, tok_a, h_a, t_a)
        fire_gather(tok_a, word_a, gsem_a)

        def half(c, cur, nxt):
            tok_c, h_c, t_c, word_c, tail_c, gsem_c, wsem_c = cur
            tok_n, h_n, t_n, word_n, tail_n, gsem_n, wsem_n = nxt

            has_next = c + 1 < n_chunk

            @pl.when(has_next)
            def _():
                load_idx(c + 1, tok_n, h_n, t_n)

            wait_gather(tok_c, word_c, gsem_c)

            @pl.when(has_next)
            def _():
                @pl.when(c >= 1)
                def _():
                    wait_write(c - 1, word_n, tail_n, wsem_n)
                fire_gather(tok_n, word_n, gsem_n)

            fill(h_c, t_c, word_c, tail_c)
            fire_write(c, word_c, tail_c, wsem_c)

        def pair(k, _):
            c = k * 2
            half(c, bufs[0], bufs[1])
            half(c + 1, bufs[1], bufs[0])
            return 0

        lax.fori_loop(0, n_chunk // 2, pair, 0)
        wait_write(n_chunk - 2, word_a, tail_a, wsem_a)
        wait_write(n_chunk - 1, word_b, tail_b, wsem_b)

    return sc_embed


def kernel(token, h_pos, t_pos, word_emb, h_pos_emb, t_pos_emb):
    b, ns, l = token.shape
    n_rows = b * ns * l
    vocab = word_emb.shape[0]
    pos_vocab = h_pos_emb.shape[0]

    wpad = jnp.concatenate(
        [word_emb, jnp.zeros((vocab, _PAD_DIM - _WORD_DIM), jnp.float32)],
        axis=1)
    tok2 = token.reshape(n_rows // _IB, _IB)
    hflat = h_pos.reshape(n_rows)
    tflat = t_pos.reshape(n_rows)

    out = _build(n_rows, pos_vocab)(
        wpad, tok2, hflat, tflat,
        h_pos_emb.reshape(-1), t_pos_emb.reshape(-1))
    return out.reshape(b, ns, l, _OUT_DIM)


# speedup vs baseline: 9.1751x; 1.0516x over previous
import functools

import jax
import jax.numpy as jnp
from jax import lax
from jax.experimental import pallas as pl
from jax.experimental.pallas import tpu as pltpu
from jax.experimental.pallas import tpu_sc as plsc

_WORD_DIM = 50
_POS_DIM = 5
_OUT_DIM = 60
_PAD_DIM = 64
_CPY_DIM = 56
_NC, _NSUB = 2, 16
_NW = _NC * _NSUB
_TB = 512
_IB = 128


@functools.lru_cache(maxsize=None)
def _build(n_rows: int, pos_vocab: int):
    n_blk = _TB // _IB
    rw = n_rows // _NW
    n_chunk = rw // _TB
    assert rw * _NW == n_rows and n_chunk * _TB == rw
    assert n_chunk % 2 == 0

    mesh = plsc.VectorSubcoreMesh(
        core_axis_name="c", subcore_axis_name="s",
        num_cores=_NC, num_subcores=_NSUB)

    tail_w = _OUT_DIM - _CPY_DIM

    @functools.partial(
        pl.kernel,
        out_type=jax.ShapeDtypeStruct((n_rows, _OUT_DIM), jnp.float32),
        mesh=mesh,
        compiler_params=pltpu.CompilerParams(
            needs_layout_passes=False, use_tc_tiling_on_sc=False),
        scratch_types=[
            pltpu.VMEM((n_blk, _IB), jnp.int32),
            pltpu.VMEM((n_blk, _IB), jnp.int32),
            pltpu.VMEM((_TB,), jnp.int32),
            pltpu.VMEM((_TB,), jnp.int32),
            pltpu.VMEM((_TB,), jnp.int32),
            pltpu.VMEM((_TB,), jnp.int32),
            pltpu.VMEM((_TB, _PAD_DIM), jnp.float32),
            pltpu.VMEM((_TB, _PAD_DIM), jnp.float32),
            pltpu.VMEM((_TB, tail_w), jnp.float32),
            pltpu.VMEM((_TB, tail_w), jnp.float32),
            pltpu.VMEM((pos_vocab * _POS_DIM,), jnp.float32),
            pltpu.VMEM((pos_vocab * _POS_DIM,), jnp.float32),
            pltpu.SemaphoreType.DMA,
            pltpu.SemaphoreType.DMA,
            pltpu.SemaphoreType.DMA,
            pltpu.SemaphoreType.DMA,
            pltpu.SemaphoreType.DMA,
            pltpu.SemaphoreType.DMA,
        ],
    )
    def sc_embed(wpad, tok2, hflat, tflat, htab, ttab, out,
                 tok_a, tok_b, h_a, h_b, t_a, t_b,
                 word_a, word_b, tail_a, tail_b, htab_v, ttab_v,
                 gsem_a, gsem_b, wsem_a, wsem_b, isem_a, isem_b):
        wid = lax.axis_index("s") * _NC + lax.axis_index("c")
        pltpu.sync_copy(htab, htab_v)
        pltpu.sync_copy(ttab, ttab_v)
        blk0 = wid * (rw // _IB)

        def idx_copies(c, tok_v, h_v, t_v, isem):
            blk = blk0 + c * n_blk
            row0 = blk * _IB
            return (
                (tok2.at[pl.ds(blk, n_blk)], tok_v, isem),
                (hflat.at[pl.ds(row0, _TB)], h_v, isem),
                (tflat.at[pl.ds(row0, _TB)], t_v, isem),
            )

        def fire_idx(c, tok_v, h_v, t_v, isem):
            for src, dst, sm in idx_copies(c, tok_v, h_v, t_v, isem):
                pltpu.async_copy(src, dst, sm)

        def wait_idx(c, tok_v, h_v, t_v, isem):
            for src, dst, sm in idx_copies(c, tok_v, h_v, t_v, isem):
                pltpu.make_async_copy(src, dst, sm).wait()

        def fire_gather(tok_v, word_v, gsem):
            for j in range(n_blk):
                pltpu.async_copy(wpad.at[tok_v.at[j]],
                                 word_v.at[pl.ds(j * _IB, _IB)], gsem)

        def wait_gather(tok_v, word_v, gsem):
            for j in range(n_blk):
                pltpu.make_async_copy(
                    wpad.at[tok_v.at[j]],
                    word_v.at[pl.ds(j * _IB, _IB)], gsem).wait()

        def fill(h_v, t_v, word_v, tail_v):
            def body(i, _):
                h5 = h_v[pl.ds(i * 16, 16)] * _POS_DIM
                t5 = t_v[pl.ds(i * 16, 16)] * _POS_DIM
                rows = i * 16 + lax.iota(jnp.int32, 16)
                for d in range(_POS_DIM):
                    hv = plsc.load_gather(htab_v, [h5 + d])
                    plsc.store_scatter(
                        word_v,
                        [rows, jnp.full((16,), _WORD_DIM + d, jnp.int32)], hv)
                for d in range(_POS_DIM):
                    tv = plsc.load_gather(ttab_v, [t5 + d])
                    if d == 0:
                        plsc.store_scatter(
                            word_v,
                            [rows,
                             jnp.full((16,), _WORD_DIM + _POS_DIM, jnp.int32)],
                            tv)
                    else:
                        plsc.store_scatter(
                            tail_v,
                            [rows, jnp.full((16,), d - 1, jnp.int32)], tv)
                return 0

            lax.fori_loop(0, _TB // 16, body, 0)

        def out_slices(c):
            row0 = (blk0 + c * n_blk) * _IB
            return (out.at[pl.ds(row0, _TB), pl.ds(0, _CPY_DIM)],
                    out.at[pl.ds(row0, _TB), pl.ds(_CPY_DIM, tail_w)])

        def fire_write(c, word_v, tail_v, wsem):
            main_dst, tail_dst = out_slices(c)
            pltpu.async_copy(word_v.at[:, pl.ds(0, _CPY_DIM)], main_dst, wsem)
            pltpu.async_copy(tail_v, tail_dst, wsem)

        def wait_write(c, word_v, tail_v, wsem):
            main_dst, tail_dst = out_slices(c)
            pltpu.make_async_copy(
                word_v.at[:, pl.ds(0, _CPY_DIM)], main_dst, wsem).wait()
            pltpu.make_async_copy(tail_v, tail_dst, wsem).wait()

        bufs = (
            (tok_a, h_a, t_a, word_a, tail_a, gsem_a, wsem_a, isem_a),
            (tok_b, h_b, t_b, word_b, tail_b, gsem_b, wsem_b, isem_b),
        )

        fire_idx(0, tok_a, h_a, t_a, isem_a)
        wait_idx(0, tok_a, h_a, t_a, isem_a)
        fire_gather(tok_a, word_a, gsem_a)
        fire_idx(1, tok_b, h_b, t_b, isem_b)

        def half(c, cur, nxt):
            tok_c, h_c, t_c, word_c, tail_c, gsem_c, wsem_c, isem_c = cur
            tok_n, h_n, t_n, word_n, tail_n, gsem_n, wsem_n, isem_n = nxt

            has_next = c + 1 < n_chunk

            wait_gather(tok_c, word_c, gsem_c)

            @pl.when(has_next)
            def _():
                @pl.when(c >= 1)
                def _():
                    wait_write(c - 1, word_n, tail_n, wsem_n)
                wait_idx(c + 1, tok_n, h_n, t_n, isem_n)
                fire_gather(tok_n, word_n, gsem_n)

            fill(h_c, t_c, word_c, tail_c)
            fire_write(c, word_c, tail_c, wsem_c)

            @pl.when(c + 2 < n_chunk)
            def _():
                fire_idx(c + 2, tok_c, h_c, t_c, isem_c)

        def pair(k, _):
            c = k * 2
            half(c, bufs[0], bufs[1])
            half(c + 1, bufs[1], bufs[0])
            return 0

        lax.fori_loop(0, n_chunk // 2, pair, 0)
        wait_write(n_chunk - 2, word_a, tail_a, wsem_a)
        wait_write(n_chunk - 1, word_b, tail_b, wsem_b)

    return sc_embed


def kernel(token, h_pos, t_pos, word_emb, h_pos_emb, t_pos_emb):
    b, ns, l = token.shape
    n_rows = b * ns * l
    vocab = word_emb.shape[0]
    pos_vocab = h_pos_emb.shape[0]

    wpad = jnp.concatenate(
        [word_emb, jnp.zeros((vocab, _PAD_DIM - _WORD_DIM), jnp.float32)],
        axis=1)
    tok2 = token.reshape(n_rows // _IB, _IB)
    hflat = h_pos.reshape(n_rows)
    tflat = t_pos.reshape(n_rows)

    out = _build(n_rows, pos_vocab)(
        wpad, tok2, hflat, tflat,
        h_pos_emb.reshape(-1), t_pos_emb.reshape(-1))
    return out.reshape(b, ns, l, _OUT_DIM)
